# baseline (speedup 1.0000x reference)
import functools

import jax
import jax.numpy as jnp
from jax import lax
from jax.experimental import pallas as pl
from jax.experimental.pallas import tpu as pltpu
from jax.experimental.pallas import tpu_sc as plsc

VOCAB = 1000
VPAD = 1024
BATCH = 1024
SEQ = 200
NUM_WORKERS = 32
BB = 128
VB = 128
KBLOCKS = VPAD // VB
N_BLOCKS = SEQ * (BATCH // BB)
BLOCKS_PER_W = N_BLOCKS // NUM_WORKERS
UNITS_PER_W = BLOCKS_PER_W * KBLOCKS
LAST_ROWS = VOCAB - 7 * VB


def _emb_body(xt_hbm, tabr_hbm, out_hbm, idxb, idxv, g0, g1, t0, t1,
              isem, gsem, wsem):
    wid = lax.axis_index("s") * 2 + lax.axis_index("c")
    lanes = lax.iota(jnp.int32, 16)
    g16 = [16 * g + lanes for g in range(8)]

    def unit_ids(u):
        blk = wid * BLOCKS_PER_W + u // KBLOCKS
        return blk // 8, blk % 8, u % KBLOCKS

    def prep_idx(u):
        s, bt, _ = unit_ids(u)
        pltpu.sync_copy(xt_hbm.at[pl.ds(s * BATCH + bt * BB, BB)], idxb)
        for g in range(8):
            vg = idxb[pl.ds(16 * g, 16)]
            for k in range(KBLOCKS):
                idxv[k, pl.ds(16 * g, 16)] = vg + 1000 * k

    def gather(u, gbuf):
        _, _, k = unit_ids(u)
        pltpu.async_copy(tabr_hbm.at[idxv.at[k]], gbuf, gsem)

    def drain_gather(gbuf):
        pltpu.make_async_copy(tabr_hbm.at[pl.ds(0, BB)], gbuf, gsem).wait()

    def transpose(gbuf, tbuf):
        def vrow(v, carry):
            colv = jnp.full((16,), v, jnp.int32)
            for g in range(8):
                tbuf[v, pl.ds(16 * g, 16)] = plsc.load_gather(
                    gbuf, [g16[g], colv])
            return carry

        lax.fori_loop(0, VB, vrow, 0)

    def write(u, tbuf):
        s, bt, k = unit_ids(u)

        @pl.when(k < KBLOCKS - 1)
        def _():
            pltpu.async_copy(
                tbuf, out_hbm.at[s, pl.ds(k * VB, VB), pl.ds(bt * BB, BB)],
                wsem)

        @pl.when(k == KBLOCKS - 1)
        def _():
            pltpu.async_copy(
                tbuf.at[pl.ds(0, LAST_ROWS)],
                out_hbm.at[s, pl.ds(7 * VB, LAST_ROWS), pl.ds(bt * BB, BB)],
                wsem)

    def wait_write(u):
        _, _, k = unit_ids(u)

        @pl.when(k < KBLOCKS - 1)
        def _():
            pltpu.make_async_copy(
                tabr_hbm.at[pl.ds(0, VB)], t0, wsem).wait()

        @pl.when(k == KBLOCKS - 1)
        def _():
            pltpu.make_async_copy(
                tabr_hbm.at[pl.ds(0, LAST_ROWS)],
                t0.at[pl.ds(0, LAST_ROWS)], wsem).wait()

    prep_idx(0)
    gather(0, g0)

    def body(t, carry):
        u0 = 2 * t
        gather(u0 + 1, g1)
        drain_gather(g0)
        transpose(g0, t0)
        write(u0, t0)
        drain_gather(g1)
        transpose(g1, t1)
        write(u0 + 1, t1)
        wait_write(u0)

        @pl.when(jnp.logical_and((u0 + 2) % KBLOCKS == 0,
                                 t + 1 < UNITS_PER_W // 2))
        def _():
            prep_idx(u0 + 2)

        @pl.when(t + 1 < UNITS_PER_W // 2)
        def _():
            gather(u0 + 2, g0)

        wait_write(u0 + 1)
        return carry

    lax.fori_loop(0, UNITS_PER_W // 2, body, 0)


@jax.jit
def _emb_call(xt_flat, tabr):
    mesh = plsc.VectorSubcoreMesh(core_axis_name="c", subcore_axis_name="s")
    f = functools.partial(
        pl.kernel,
        mesh=mesh,
        out_type=jax.ShapeDtypeStruct((SEQ, VOCAB, BATCH), jnp.float32),
        scratch_types=[
            pltpu.VMEM((BB,), jnp.int32),
            pltpu.VMEM((KBLOCKS, BB), jnp.int32),
            pltpu.VMEM((BB, VB), jnp.float32),
            pltpu.VMEM((BB, VB), jnp.float32),
            pltpu.VMEM((VB, BB), jnp.float32),
            pltpu.VMEM((VB, BB), jnp.float32),
            pltpu.SemaphoreType.DMA,
            pltpu.SemaphoreType.DMA,
            pltpu.SemaphoreType.DMA,
        ],
        compiler_params=pltpu.CompilerParams(needs_layout_passes=False),
    )(_emb_body)
    return f(xt_flat, tabr)


def kernel(x, table):
    xt_flat = x.T.reshape(SEQ * BATCH).astype(jnp.int32)
    tp = jnp.pad(table, ((0, 0), (0, VPAD - VOCAB)))
    tabr = tp.reshape(VOCAB, KBLOCKS, VB).transpose(1, 0, 2).reshape(
        KBLOCKS * VOCAB, VB)
    out_t = _emb_call(xt_flat, tabr)
    return jnp.transpose(out_t, (2, 0, 1))
